# baseline (speedup 1.0000x reference)
import functools

import jax
import jax.numpy as jnp
from jax import lax
from jax.experimental import pallas as pl
from jax.experimental.pallas import tpu as pltpu
from jax.experimental.pallas import tpu_sc as plsc

NC = 2
NS = 16
NW = NC * NS
CB = 64
NB = 4
NI = 8
F0 = 0.675


def _chunk_split(e):
    t = -(-e // CB)
    c0 = -(-int(t * F0) // NS)
    c0 += (-c0) % NB
    c1 = max(NI, -(-(t - NS * c0) // NS))
    c1 += (-c1) % NB
    return c0, c1


def _sc_aggregate(x, edges_r, n_pad, c0, c1):
    n, di = x.shape
    z_ch = n_pad // CB
    z_per_tile = (z_ch + NS - 1) // NS
    o_full = n // CB
    o_rem = n - o_full * CB
    o_per_tile = (o_full + NS - 1) // NS

    mesh = plsc.VectorSubcoreMesh(core_axis_name="c", subcore_axis_name="s")

    @functools.partial(
        pl.kernel,
        out_type=jax.ShapeDtypeStruct((NC, n, di), jnp.float32),
        mesh=mesh,
        scratch_types=[
            pltpu.VMEM((NI, 2, CB), jnp.int32),
            pltpu.VMEM((NB, CB, di), jnp.float32),
            pltpu.SemaphoreType.DMA,
            pltpu.SemaphoreType.DMA,
            pltpu.SemaphoreType.DMA,
            pltpu.SemaphoreType.DMA,
            pltpu.SemaphoreType.DMA,
            pltpu.SemaphoreType.DMA,
            pltpu.SemaphoreType.DMA,
            pltpu.SemaphoreType.DMA,
            pltpu.SemaphoreType.DMA,
            pltpu.SemaphoreType.DMA,
            pltpu.VMEM_SHARED((n_pad, di), jnp.float32),
        ],
    )
    def agg_kernel(edges_hbm, x_hbm, out_hbm, idx_v, rows_v,
                   g0, g1, g2, g3, s0, s1, s2, s3, i0, i1, acc_sh):
        cid = lax.axis_index("c")
        sid = lax.axis_index("s")
        gsems = (g0, g1, g2, g3)
        ssems = (s0, s1, s2, s3)
        isems = (i0, i1)

        cnt = lax.select(cid == 0, jnp.int32(c0), jnp.int32(c1))
        base = cid * (NS * c0) + sid * cnt

        def fill_zero(i, _):
            rows_v[0, i // 8, pl.ds((i % 8) * 16, 16)] = jnp.zeros(
                (16,), jnp.float32)
            return 0
        lax.fori_loop(0, CB * (di // 16), fill_zero, 0)

        def zero_chunk(t, _):
            j = sid + NS * t

            @pl.when(j < z_ch)
            def _():
                pltpu.sync_copy(rows_v.at[0], acc_sh.at[pl.ds(j * CB, CB)])
            return 0
        lax.fori_loop(0, z_per_tile, zero_chunk, 0)
        plsc.subcore_barrier()

        def idx_load(j, p):
            pltpu.async_copy(edges_hbm.at[base + j], idx_v.at[j % NI],
                             isems[p])

        def wait_idx(p):
            pltpu.make_async_copy(edges_hbm.at[0], idx_v.at[0],
                                  isems[p]).wait()

        def gather(q, b):
            pltpu.async_copy(x_hbm.at[idx_v.at[q, 0]], rows_v.at[b],
                             gsems[b])

        def wait_gather(b):
            pltpu.make_async_copy(x_hbm.at[pl.ds(0, CB)], rows_v.at[b],
                                  gsems[b]).wait()

        def scatter(q, b):
            pltpu.async_copy(rows_v.at[b], acc_sh.at[idx_v.at[q, 1]],
                             ssems[b], add=True)

        def wait_scatter(b):
            pltpu.make_async_copy(rows_v.at[b], acc_sh.at[pl.ds(0, CB)],
                                  ssems[b]).wait()

        for j in range(NI):
            idx_load(jnp.int32(j), j % 2)
        for b in range(NB):
            wait_idx(b % 2)
            gather(jnp.int32(b), b)

        def edge_group(t, _):
            a = NB * t
            for b in range(NB):
                j = a + b
                wait_gather(b)
                scatter(j % NI, b)
            for b in range(NB):
                j = a + b
                wait_scatter(b)

                @pl.when(j + NI < cnt)
                def _():
                    idx_load(j + NI, b % 2)

                @pl.when(j + NB < cnt)
                def _():
                    wait_idx(b % 2)
                    gather((j + NB) % NI, b)
            return 0
        lax.fori_loop(0, cnt // NB, edge_group, 0)
        plsc.subcore_barrier()

        def out_chunk(t, _):
            j = sid + NS * t

            @pl.when(j < o_full)
            def _():
                pltpu.sync_copy(acc_sh.at[pl.ds(j * CB, CB)], rows_v.at[0])
                pltpu.sync_copy(rows_v.at[0],
                                out_hbm.at[cid].at[pl.ds(j * CB, CB)])
            return 0
        lax.fori_loop(0, o_per_tile, out_chunk, 0)

        if o_rem:
            @pl.when(sid == 0)
            def _():
                pltpu.sync_copy(acc_sh.at[pl.ds(o_full * CB, o_rem)],
                                rows_v.at[0].at[pl.ds(0, o_rem)])
                pltpu.sync_copy(rows_v.at[0].at[pl.ds(0, o_rem)],
                                out_hbm.at[cid].at[pl.ds(o_full * CB, o_rem)])

    return agg_kernel(edges_r, x)


def _mlp_body(x_ref, agg_ref, eps_ref, w1_ref, b1_ref, g1_ref, be1_ref,
              w2_ref, b2_ref, g2_ref, be2_ref, o_ref):
    h = x_ref[...] + eps_ref[...] * x_ref[...] + agg_ref[0] + agg_ref[1]
    h = jnp.dot(h, w1_ref[...], preferred_element_type=jnp.float32) + b1_ref[...]
    mu = jnp.mean(h, axis=0, keepdims=True)
    var = jnp.mean((h - mu) * (h - mu), axis=0, keepdims=True)
    h = g1_ref[...] * (h - mu) * lax.rsqrt(var + 1e-5) + be1_ref[...]
    h = jnp.maximum(h, 0.0)
    h = jnp.dot(h, w2_ref[...], preferred_element_type=jnp.float32) + b2_ref[...]
    mu2 = jnp.mean(h, axis=0, keepdims=True)
    var2 = jnp.mean((h - mu2) * (h - mu2), axis=0, keepdims=True)
    h = g2_ref[...] * (h - mu2) * lax.rsqrt(var2 + 1e-5) + be2_ref[...]
    o_ref[...] = jnp.maximum(h, 0.0)


def kernel(x, edge_index, eps, W1, b1, gamma1, beta1, W2, b2, gamma2, beta2):
    n, di = x.shape
    e = edge_index.shape[1]

    c0, c1 = _chunk_split(e)
    t_pad = NS * (c0 + c1)
    e_pad = t_pad * CB
    n_pad = -(-(n + 1) // CB) * CB
    dst = edge_index[0].astype(jnp.int32)
    src = edge_index[1].astype(jnp.int32)
    pad = e_pad - e
    if pad:
        trash = n + jnp.arange(pad, dtype=jnp.int32) % jnp.int32(n_pad - n)
        src = jnp.concatenate([src, jnp.zeros((pad,), jnp.int32)])
        dst = jnp.concatenate([dst, trash])
    edges_r = jnp.concatenate([src.reshape(t_pad, 1, CB),
                               dst.reshape(t_pad, 1, CB)], axis=1)

    agg = _sc_aggregate(x, edges_r, n_pad, c0, c1)

    out = pl.pallas_call(
        _mlp_body,
        out_shape=jax.ShapeDtypeStruct((n, di), jnp.float32),
    )(x, agg, eps.reshape(1, 1), W1, b1.reshape(1, di),
      gamma1.reshape(1, di), beta1.reshape(1, di), W2, b2.reshape(1, di),
      gamma2.reshape(1, di), beta2.reshape(1, di))
    return out


# speedup vs baseline: 1.0658x; 1.0658x over previous
import functools

import jax
import jax.numpy as jnp
from jax import lax
from jax.experimental import pallas as pl
from jax.experimental.pallas import tpu as pltpu
from jax.experimental.pallas import tpu_sc as plsc

NC = 2
NS = 16
NW = NC * NS
CB = 64
NB = 4
NI = 8
F0 = 0.80


def _chunk_split(e):
    t = -(-e // CB)
    c0 = -(-int(t * F0) // NS)
    c0 += (-c0) % NB
    c1 = max(NI, -(-(t - NS * c0) // NS))
    c1 += (-c1) % NB
    return c0, c1


def _sc_aggregate(x, edges_r, n_pad, c0, c1):
    n, di = x.shape
    z_ch = n_pad // CB
    z_per_tile = (z_ch + NS - 1) // NS
    o_full = n // CB
    o_rem = n - o_full * CB
    o_per_tile = (o_full + NS - 1) // NS

    mesh = plsc.VectorSubcoreMesh(core_axis_name="c", subcore_axis_name="s")

    @functools.partial(
        pl.kernel,
        out_type=jax.ShapeDtypeStruct((NC, n, di), jnp.float32),
        mesh=mesh,
        scratch_types=[
            pltpu.VMEM((NI, 2, CB), jnp.int32),
            pltpu.VMEM((NB, CB, di), jnp.float32),
            pltpu.SemaphoreType.DMA,
            pltpu.SemaphoreType.DMA,
            pltpu.SemaphoreType.DMA,
            pltpu.SemaphoreType.DMA,
            pltpu.SemaphoreType.DMA,
            pltpu.SemaphoreType.DMA,
            pltpu.SemaphoreType.DMA,
            pltpu.SemaphoreType.DMA,
            pltpu.SemaphoreType.DMA,
            pltpu.SemaphoreType.DMA,
            pltpu.VMEM_SHARED((n_pad, di), jnp.float32),
        ],
    )
    def agg_kernel(edges_hbm, x_hbm, out_hbm, idx_v, rows_v,
                   g0, g1, g2, g3, s0, s1, s2, s3, i0, i1, acc_sh):
        cid = lax.axis_index("c")
        sid = lax.axis_index("s")
        gsems = (g0, g1, g2, g3)
        ssems = (s0, s1, s2, s3)
        isems = (i0, i1)

        cnt = lax.select(cid == 0, jnp.int32(c0), jnp.int32(c1))
        base = cid * (NS * c0) + sid * cnt

        def fill_zero(i, _):
            rows_v[0, i // 8, pl.ds((i % 8) * 16, 16)] = jnp.zeros(
                (16,), jnp.float32)
            return 0
        lax.fori_loop(0, CB * (di // 16), fill_zero, 0)

        def zero_chunk(t, _):
            j = sid + NS * t

            @pl.when(j < z_ch)
            def _():
                pltpu.sync_copy(rows_v.at[0], acc_sh.at[pl.ds(j * CB, CB)])
            return 0
        lax.fori_loop(0, z_per_tile, zero_chunk, 0)
        plsc.subcore_barrier()

        def idx_load(j, p):
            pltpu.async_copy(edges_hbm.at[base + j], idx_v.at[j % NI],
                             isems[p])

        def wait_idx(p):
            pltpu.make_async_copy(edges_hbm.at[0], idx_v.at[0],
                                  isems[p]).wait()

        def gather(q, b):
            pltpu.async_copy(x_hbm.at[idx_v.at[q, 0]], rows_v.at[b],
                             gsems[b])

        def wait_gather(b):
            pltpu.make_async_copy(x_hbm.at[pl.ds(0, CB)], rows_v.at[b],
                                  gsems[b]).wait()

        def scatter(q, b):
            pltpu.async_copy(rows_v.at[b], acc_sh.at[idx_v.at[q, 1]],
                             ssems[b], add=True)

        def wait_scatter(b):
            pltpu.make_async_copy(rows_v.at[b], acc_sh.at[pl.ds(0, CB)],
                                  ssems[b]).wait()

        for j in range(NI):
            idx_load(jnp.int32(j), j % 2)
        for b in range(NB):
            wait_idx(b % 2)
            gather(jnp.int32(b), b)

        def edge_group(t, _):
            a = NB * t
            for b in range(NB):
                j = a + b
                wait_gather(b)
                scatter(j % NI, b)
            for b in range(NB):
                j = a + b
                wait_scatter(b)

                @pl.when(j + NI < cnt)
                def _():
                    idx_load(j + NI, b % 2)

                @pl.when(j + NB < cnt)
                def _():
                    wait_idx(b % 2)
                    gather((j + NB) % NI, b)
            return 0
        lax.fori_loop(0, cnt // NB, edge_group, 0)
        plsc.subcore_barrier()

        def out_chunk(t, _):
            j = sid + NS * t

            @pl.when(j < o_full)
            def _():
                pltpu.sync_copy(acc_sh.at[pl.ds(j * CB, CB)], rows_v.at[0])
                pltpu.sync_copy(rows_v.at[0],
                                out_hbm.at[cid].at[pl.ds(j * CB, CB)])
            return 0
        lax.fori_loop(0, o_per_tile, out_chunk, 0)

        if o_rem:
            @pl.when(sid == 0)
            def _():
                pltpu.sync_copy(acc_sh.at[pl.ds(o_full * CB, o_rem)],
                                rows_v.at[0].at[pl.ds(0, o_rem)])
                pltpu.sync_copy(rows_v.at[0].at[pl.ds(0, o_rem)],
                                out_hbm.at[cid].at[pl.ds(o_full * CB, o_rem)])

    return agg_kernel(edges_r, x)


def _mlp_body(x_ref, agg_ref, eps_ref, w1_ref, b1_ref, g1_ref, be1_ref,
              w2_ref, b2_ref, g2_ref, be2_ref, o_ref):
    h = x_ref[...] + eps_ref[...] * x_ref[...] + agg_ref[0] + agg_ref[1]
    h = jnp.dot(h, w1_ref[...], preferred_element_type=jnp.float32) + b1_ref[...]
    mu = jnp.mean(h, axis=0, keepdims=True)
    var = jnp.mean((h - mu) * (h - mu), axis=0, keepdims=True)
    h = g1_ref[...] * (h - mu) * lax.rsqrt(var + 1e-5) + be1_ref[...]
    h = jnp.maximum(h, 0.0)
    h = jnp.dot(h, w2_ref[...], preferred_element_type=jnp.float32) + b2_ref[...]
    mu2 = jnp.mean(h, axis=0, keepdims=True)
    var2 = jnp.mean((h - mu2) * (h - mu2), axis=0, keepdims=True)
    h = g2_ref[...] * (h - mu2) * lax.rsqrt(var2 + 1e-5) + be2_ref[...]
    o_ref[...] = jnp.maximum(h, 0.0)


def kernel(x, edge_index, eps, W1, b1, gamma1, beta1, W2, b2, gamma2, beta2):
    n, di = x.shape
    e = edge_index.shape[1]

    c0, c1 = _chunk_split(e)
    t_pad = NS * (c0 + c1)
    e_pad = t_pad * CB
    n_pad = -(-(n + 1) // CB) * CB
    dst = edge_index[0].astype(jnp.int32)
    src = edge_index[1].astype(jnp.int32)
    pad = e_pad - e
    if pad:
        trash = n + jnp.arange(pad, dtype=jnp.int32) % jnp.int32(n_pad - n)
        src = jnp.concatenate([src, jnp.zeros((pad,), jnp.int32)])
        dst = jnp.concatenate([dst, trash])
    edges_r = jnp.concatenate([src.reshape(t_pad, 1, CB),
                               dst.reshape(t_pad, 1, CB)], axis=1)

    agg = _sc_aggregate(x, edges_r, n_pad, c0, c1)

    out = pl.pallas_call(
        _mlp_body,
        out_shape=jax.ShapeDtypeStruct((n, di), jnp.float32),
    )(x, agg, eps.reshape(1, 1), W1, b1.reshape(1, di),
      gamma1.reshape(1, di), beta1.reshape(1, di), W2, b2.reshape(1, di),
      gamma2.reshape(1, di), beta2.reshape(1, di))
    return out


# speedup vs baseline: 1.0815x; 1.0147x over previous
import functools

import jax
import jax.numpy as jnp
from jax import lax
from jax.experimental import pallas as pl
from jax.experimental.pallas import tpu as pltpu
from jax.experimental.pallas import tpu_sc as plsc

NC = 2
NS = 16
NW = NC * NS
CB = 64
NB = 4
NI = 8
F0 = 0.80


def _chunk_split(e):
    t = -(-e // CB)
    c0 = -(-int(t * F0) // NS)
    c0 += (-c0) % NB
    c1 = max(NI, -(-(t - NS * c0) // NS))
    c1 += (-c1) % NB
    return c0, c1


def _sc_aggregate(x, src_r, dst_r, n_pad, c0, c1):
    n, di = x.shape
    z_ch = n_pad // CB
    z_per_tile = (z_ch + NS - 1) // NS
    o_full = n // CB
    o_rem = n - o_full * CB
    o_per_tile = (o_full + NS - 1) // NS

    mesh = plsc.VectorSubcoreMesh(core_axis_name="c", subcore_axis_name="s")

    @functools.partial(
        pl.kernel,
        out_type=jax.ShapeDtypeStruct((NC, n, di), jnp.float32),
        mesh=mesh,
        scratch_types=[
            pltpu.VMEM((NI, CB), jnp.int32),
            pltpu.VMEM((NI, CB), jnp.int32),
            pltpu.VMEM((NB, CB, di), jnp.float32),
            pltpu.SemaphoreType.DMA,
            pltpu.SemaphoreType.DMA,
            pltpu.SemaphoreType.DMA,
            pltpu.SemaphoreType.DMA,
            pltpu.SemaphoreType.DMA,
            pltpu.SemaphoreType.DMA,
            pltpu.SemaphoreType.DMA,
            pltpu.SemaphoreType.DMA,
            pltpu.SemaphoreType.DMA,
            pltpu.SemaphoreType.DMA,
            pltpu.SemaphoreType.DMA,
            pltpu.SemaphoreType.DMA,
            pltpu.VMEM_SHARED((n_pad, di), jnp.float32),
        ],
    )
    def agg_kernel(src_hbm, dst_hbm, x_hbm, out_hbm, sidx_v, didx_v, rows_v,
                   g0, g1, g2, g3, s0, s1, s2, s3, i0, i1, i2, i3, acc_sh):
        cid = lax.axis_index("c")
        sid = lax.axis_index("s")
        gsems = (g0, g1, g2, g3)
        ssems = (s0, s1, s2, s3)
        isems = (i0, i1)
        jsems = (i2, i3)

        cnt = lax.select(cid == 0, jnp.int32(c0), jnp.int32(c1))
        base = cid * (NS * c0) + sid * cnt

        def fill_zero(i, _):
            rows_v[0, i // 8, pl.ds((i % 8) * 16, 16)] = jnp.zeros(
                (16,), jnp.float32)
            return 0
        lax.fori_loop(0, CB * (di // 16), fill_zero, 0)

        def zero_chunk(t, _):
            j = sid + NS * t

            @pl.when(j < z_ch)
            def _():
                pltpu.sync_copy(rows_v.at[0], acc_sh.at[pl.ds(j * CB, CB)])
            return 0
        lax.fori_loop(0, z_per_tile, zero_chunk, 0)
        plsc.subcore_barrier()

        def idx_load(j, p):
            pltpu.async_copy(src_hbm.at[base + j], sidx_v.at[j % NI],
                             isems[p])
            pltpu.async_copy(dst_hbm.at[base + j], didx_v.at[j % NI],
                             jsems[p])

        def wait_idx(p):
            pltpu.make_async_copy(src_hbm.at[0], sidx_v.at[0],
                                  isems[p]).wait()
            pltpu.make_async_copy(dst_hbm.at[0], didx_v.at[0],
                                  jsems[p]).wait()

        def gather(q, b):
            pltpu.async_copy(x_hbm.at[sidx_v.at[q]], rows_v.at[b],
                             gsems[b])

        def wait_gather(b):
            pltpu.make_async_copy(x_hbm.at[pl.ds(0, CB)], rows_v.at[b],
                                  gsems[b]).wait()

        def scatter(q, b):
            pltpu.async_copy(rows_v.at[b], acc_sh.at[didx_v.at[q]],
                             ssems[b], add=True)

        def wait_scatter(b):
            pltpu.make_async_copy(rows_v.at[b], acc_sh.at[pl.ds(0, CB)],
                                  ssems[b]).wait()

        for j in range(NI):
            idx_load(jnp.int32(j), j % 2)
        for b in range(NB):
            wait_idx(b % 2)
            gather(jnp.int32(b), b)

        def edge_group(t, _):
            a = NB * t
            for b in range(NB):
                j = a + b
                wait_gather(b)
                scatter(j % NI, b)
            for b in range(NB):
                j = a + b
                wait_scatter(b)

                @pl.when(j + NI < cnt)
                def _():
                    idx_load(j + NI, b % 2)

                @pl.when(j + NB < cnt)
                def _():
                    wait_idx(b % 2)
                    gather((j + NB) % NI, b)
            return 0
        lax.fori_loop(0, cnt // NB, edge_group, 0)
        plsc.subcore_barrier()

        def out_chunk(t, _):
            j = sid + NS * t

            @pl.when(j < o_full)
            def _():
                pltpu.sync_copy(acc_sh.at[pl.ds(j * CB, CB)], rows_v.at[0])
                pltpu.sync_copy(rows_v.at[0],
                                out_hbm.at[cid].at[pl.ds(j * CB, CB)])
            return 0
        lax.fori_loop(0, o_per_tile, out_chunk, 0)

        if o_rem:
            @pl.when(sid == 0)
            def _():
                pltpu.sync_copy(acc_sh.at[pl.ds(o_full * CB, o_rem)],
                                rows_v.at[0].at[pl.ds(0, o_rem)])
                pltpu.sync_copy(rows_v.at[0].at[pl.ds(0, o_rem)],
                                out_hbm.at[cid].at[pl.ds(o_full * CB, o_rem)])

    return agg_kernel(src_r, dst_r, x)


def _mlp_body(x_ref, agg_ref, eps_ref, w1_ref, b1_ref, g1_ref, be1_ref,
              w2_ref, b2_ref, g2_ref, be2_ref, o_ref):
    h = x_ref[...] + eps_ref[...] * x_ref[...] + agg_ref[0] + agg_ref[1]
    h = jnp.dot(h, w1_ref[...], preferred_element_type=jnp.float32) + b1_ref[...]
    mu = jnp.mean(h, axis=0, keepdims=True)
    var = jnp.mean((h - mu) * (h - mu), axis=0, keepdims=True)
    h = g1_ref[...] * (h - mu) * lax.rsqrt(var + 1e-5) + be1_ref[...]
    h = jnp.maximum(h, 0.0)
    h = jnp.dot(h, w2_ref[...], preferred_element_type=jnp.float32) + b2_ref[...]
    mu2 = jnp.mean(h, axis=0, keepdims=True)
    var2 = jnp.mean((h - mu2) * (h - mu2), axis=0, keepdims=True)
    h = g2_ref[...] * (h - mu2) * lax.rsqrt(var2 + 1e-5) + be2_ref[...]
    o_ref[...] = jnp.maximum(h, 0.0)


def kernel(x, edge_index, eps, W1, b1, gamma1, beta1, W2, b2, gamma2, beta2):
    n, di = x.shape
    e = edge_index.shape[1]

    c0, c1 = _chunk_split(e)
    t_pad = NS * (c0 + c1)
    e_pad = t_pad * CB
    n_pad = -(-(n + 1) // CB) * CB
    dst = edge_index[0].astype(jnp.int32)
    src = edge_index[1].astype(jnp.int32)
    pad = e_pad - e
    if pad:
        trash = n + jnp.arange(pad, dtype=jnp.int32) % jnp.int32(n_pad - n)
        src = jnp.concatenate([src, jnp.zeros((pad,), jnp.int32)])
        dst = jnp.concatenate([dst, trash])
    agg = _sc_aggregate(x, src.reshape(t_pad, CB), dst.reshape(t_pad, CB),
                        n_pad, c0, c1)

    out = pl.pallas_call(
        _mlp_body,
        out_shape=jax.ShapeDtypeStruct((n, di), jnp.float32),
    )(x, agg, eps.reshape(1, 1), W1, b1.reshape(1, di),
      gamma1.reshape(1, di), beta1.reshape(1, di), W2, b2.reshape(1, di),
      gamma2.reshape(1, di), beta2.reshape(1, di))
    return out
